# baseline (speedup 1.0000x reference)
import functools

import jax
import jax.numpy as jnp
from jax import lax
from jax.experimental import pallas as pl
from jax.experimental.pallas import tpu as pltpu
from jax.experimental.pallas import tpu_sc as plsc

_IDX_CHUNK = 128


@functools.lru_cache(maxsize=None)
def _build(B, V, D):
    info = plsc.get_sparse_core_info()
    NC, NS, L = info.num_cores, info.num_subcores, info.num_lanes
    NW = NC * NS
    assert B % NW == 0
    b_per_w = B // NW
    n_chunks = b_per_w // _IDX_CHUNK
    mesh = plsc.VectorSubcoreMesh(core_axis_name="c", subcore_axis_name="s")

    @functools.partial(
        pl.kernel,
        mesh=mesh,
        out_type=jax.ShapeDtypeStruct((D, B), jnp.float32),
        compiler_params=pltpu.CompilerParams(use_tc_tiling_on_sc=False),
        scratch_types=[
            pltpu.VMEM((n_chunks, _IDX_CHUNK), jnp.int32),
            pltpu.VMEM((n_chunks, _IDX_CHUNK), jnp.int32),
            pltpu.VMEM((D, b_per_w), jnp.float32),
            pltpu.VMEM((D, b_per_w), jnp.float32),
            pltpu.SemaphoreType.DMA,
            pltpu.SemaphoreType.DMA,
        ],
    )
    def gmf(uid_hbm, iid_hbm, ut_hbm, it_hbm, out_hbm,
            uidx_v, iidx_v, uvals_v, ivals_v, sem_u, sem_i):
        wid = lax.axis_index("s") * NC + lax.axis_index("c")
        base = wid * b_per_w
        pltpu.sync_copy(uid_hbm.at[wid], uidx_v)
        pltpu.sync_copy(iid_hbm.at[wid], iidx_v)

        def fetch(d, carry):
            for c in range(n_chunks):
                cols = pl.ds(c * _IDX_CHUNK, _IDX_CHUNK)
                pltpu.async_copy(
                    ut_hbm.at[d].at[uidx_v.at[c]], uvals_v.at[d, cols], sem_u
                )
                pltpu.async_copy(
                    it_hbm.at[d].at[iidx_v.at[c]], ivals_v.at[d, cols], sem_i
                )
            return carry

        lax.fori_loop(0, D, fetch, 0)
        pltpu.make_async_copy(
            ut_hbm.at[:, pl.ds(0, b_per_w)], uvals_v, sem_u
        ).wait()
        pltpu.make_async_copy(
            it_hbm.at[:, pl.ds(0, b_per_w)], ivals_v, sem_i
        ).wait()

        def mul_row(d, carry):
            for j in range(b_per_w // L):
                s = pl.ds(j * L, L)
                uvals_v[d, s] = uvals_v[d, s] * ivals_v[d, s]
            return carry

        lax.fori_loop(0, D, mul_row, 0)
        pltpu.sync_copy(uvals_v, out_hbm.at[:, pl.ds(base, b_per_w)])

    def run(user_ids, item_ids, user_table, item_table):
        uid = user_ids.reshape(NW, n_chunks, _IDX_CHUNK)
        iid = item_ids.reshape(NW, n_chunks, _IDX_CHUNK)
        out_t = gmf(uid, iid, user_table.T, item_table.T)
        return out_t.T

    return run


@jax.jit
def kernel(user_ids, item_ids, user_table, item_table):
    (B,) = user_ids.shape
    V, D = user_table.shape
    return _build(B, V, D)(user_ids, item_ids, user_table, item_table)


# speedup vs baseline: 7.9123x; 7.9123x over previous
import functools

import jax
import jax.numpy as jnp
from jax import lax
from jax.experimental import pallas as pl
from jax.experimental.pallas import tpu as pltpu
from jax.experimental.pallas import tpu_sc as plsc

_PACK = 4
_VB = 1024
_WAVE = 128


def _detile_body(p0_ref, p1_ref, p2_ref, p3_ref, out_ref):
    out_ref[...] = jnp.concatenate(
        [p0_ref[...].T, p1_ref[...].T, p2_ref[...].T, p3_ref[...].T], axis=1
    )


@functools.lru_cache(maxsize=None)
def _build(B, V, D):
    info = plsc.get_sparse_core_info()
    NC, NS, L = info.num_cores, info.num_subcores, info.num_lanes
    NW = NC * NS
    assert B % NW == 0 and D == 32
    b_per_w = B // NW
    n_waves = b_per_w // _WAVE
    mesh = plsc.VectorSubcoreMesh(core_axis_name="c", subcore_axis_name="s")

    grid = (V + _PACK * _VB - 1) // (_PACK * _VB)
    packed_rows = grid * _VB
    last_in_block = V // _VB
    in_specs = [
        pl.BlockSpec(
            (D, _VB),
            lambda c, t=t: (0, jnp.minimum(_PACK * c + t, last_in_block)),
        )
        for t in range(_PACK)
    ]
    detile = pl.pallas_call(
        _detile_body,
        grid=(grid,),
        in_specs=in_specs,
        out_specs=pl.BlockSpec((_VB, _PACK * D), lambda c: (c, 0)),
        out_shape=jax.ShapeDtypeStruct((packed_rows, _PACK * D), jnp.float32),
    )

    @functools.partial(
        pl.kernel,
        mesh=mesh,
        out_type=jax.ShapeDtypeStruct((D * B,), jnp.float32),
        compiler_params=pltpu.CompilerParams(
            use_tc_tiling_on_sc=True, needs_layout_passes=False
        ),
        scratch_types=[
            pltpu.VMEM((b_per_w,), jnp.int32),
            pltpu.VMEM((b_per_w,), jnp.int32),
            pltpu.VMEM((b_per_w,), jnp.int32),
            pltpu.VMEM((b_per_w,), jnp.int32),
            pltpu.VMEM((b_per_w,), jnp.int32),
            pltpu.VMEM((b_per_w,), jnp.int32),
            pltpu.VMEM((_WAVE, _PACK * D), jnp.float32),
            pltpu.VMEM((_WAVE, _PACK * D), jnp.float32),
            pltpu.VMEM((D * b_per_w,), jnp.float32),
            pltpu.SemaphoreType.DMA,
            pltpu.SemaphoreType.DMA,
            pltpu.SemaphoreType.DMA,
        ],
    )
    def gmf(uid_hbm, iid_hbm, up_hbm, ip_hbm, out_hbm,
            uids_v, iids_v, urb_v, irb_v, ulq_v, ilq_v,
            uw_v, iw_v, prod_v, sem_u, sem_i, sem_o):
        wid = lax.axis_index("s") * NC + lax.axis_index("c")
        base = wid * b_per_w
        pltpu.sync_copy(uid_hbm.at[pl.ds(base, b_per_w)], uids_v)
        pltpu.sync_copy(iid_hbm.at[pl.ds(base, b_per_w)], iids_v)

        def prep(g, carry):
            s = pl.ds(g * L, L)
            u = uids_v[s]
            i = iids_v[s]
            urb_v[s] = lax.shift_left(lax.shift_right_logical(u, 12), 10) + \
                lax.bitwise_and(u, _VB - 1)
            irb_v[s] = lax.shift_left(lax.shift_right_logical(i, 12), 10) + \
                lax.bitwise_and(i, _VB - 1)
            ulq_v[s] = lax.shift_left(
                lax.bitwise_and(lax.shift_right_logical(u, 10), _PACK - 1), 5)
            ilq_v[s] = lax.shift_left(
                lax.bitwise_and(lax.shift_right_logical(i, 10), _PACK - 1), 5)
            return carry

        lax.fori_loop(0, b_per_w // L, prep, 0)

        row_iota = lax.iota(jnp.int32, L)
        for w in range(n_waves):
            cu = pltpu.async_copy(
                up_hbm.at[urb_v.at[pl.ds(w * _WAVE, _WAVE)]], uw_v, sem_u)
            ci = pltpu.async_copy(
                ip_hbm.at[irb_v.at[pl.ds(w * _WAVE, _WAVE)]], iw_v, sem_i)
            cu.wait()
            ci.wait()

            def select(d, carry, _w=w):
                for g in range(_WAVE // L):
                    rows = row_iota + (g * L)
                    sl = pl.ds(_w * _WAVE + g * L, L)
                    uv = plsc.load_gather(uw_v, [rows, ulq_v[sl] + d])
                    iv = plsc.load_gather(iw_v, [rows, ilq_v[sl] + d])
                    prod_v[pl.ds(d * b_per_w + _w * _WAVE + g * L, L)] = uv * iv
                return carry

            lax.fori_loop(0, D, select, 0)

        def flush(d, carry):
            pltpu.async_copy(
                prod_v.at[pl.ds(d * b_per_w, b_per_w)],
                out_hbm.at[pl.ds(d * B + base, b_per_w)],
                sem_o,
            )
            return carry

        lax.fori_loop(0, D, flush, 0)
        pltpu.make_async_copy(
            out_hbm.at[pl.ds(0, D * b_per_w)], prod_v, sem_o
        ).wait()

    def run(user_ids, item_ids, user_table, item_table):
        ut = user_table.T
        it = item_table.T
        up = detile(ut, ut, ut, ut)
        ip = detile(it, it, it, it)
        out1d = gmf(user_ids, item_ids, up, ip)
        return out1d.reshape(D, B).T

    return run


@jax.jit
def kernel(user_ids, item_ids, user_table, item_table):
    (B,) = user_ids.shape
    V, D = user_table.shape
    return _build(B, V, D)(user_ids, item_ids, user_table, item_table)
